# baseline (speedup 1.0000x reference)
import jax
import jax.numpy as jnp
from jax import lax
from jax.experimental import pallas as pl
from jax.experimental.pallas import tpu as pltpu
from jax.experimental.pallas import tpu_sc as plsc

N = 10000
E = 320000
D = 128
NSUB = 16
EB = 80
EPS_DEN = 1e-6
EPS_BN = 1e-5

NW = 2 * NSUB
E_PER_W = E // NW
NBLK_W = E_PER_W // EB
N_PAD = 10240
ZROWS = 40


def _node_linear_body(h_ref, wa_ref, ba_ref, wb_ref, bb_ref, wd_ref,
                      bd_ref, we_ref, be_ref,
                      ah_ref, eh_ref, dh_ref, bh_ref):
    hv = h_ref[...]
    f32 = jnp.float32
    ah_ref[...] = jnp.dot(hv, wa_ref[...],
                          preferred_element_type=f32) + ba_ref[...]
    eh_ref[...] = jnp.dot(hv, we_ref[...],
                          preferred_element_type=f32) + be_ref[...]
    dh_ref[...] = jnp.dot(hv, wd_ref[...],
                          preferred_element_type=f32) + bd_ref[...]
    bh_ref[...] = jnp.dot(hv, wb_ref[...],
                          preferred_element_type=f32) + bb_ref[...]


def _node_linears(h, W_A, b_A, W_B, b_B, W_D, b_D, W_E, b_E):
    f32 = jnp.float32
    return pl.pallas_call(
        _node_linear_body,
        out_shape=[
            jax.ShapeDtypeStruct((N, D), f32),
            jax.ShapeDtypeStruct((N, D), f32),
            jax.ShapeDtypeStruct((N, D), f32),
            jax.ShapeDtypeStruct((N, D), f32),
        ],
    )(h, W_A, b_A.reshape(1, D), W_B, b_B.reshape(1, D), W_D,
      b_D.reshape(1, D), W_E, b_E.reshape(1, D))


CE_BLK = 2000


def _ce_body(e_ref, wc_ref, bc_ref, ce_ref):
    ce_ref[...] = jnp.dot(e_ref[...], wc_ref[...],
                          preferred_element_type=jnp.float32) + bc_ref[...]


def _ce_linear(e, W_C, b_C):
    f32 = jnp.float32
    grid = E // CE_BLK
    return pl.pallas_call(
        _ce_body,
        grid=(grid,),
        in_specs=[
            pl.BlockSpec((CE_BLK, D), lambda i: (i, 0)),
            pl.BlockSpec((D, D), lambda i: (0, 0)),
            pl.BlockSpec((1, D), lambda i: (0, 0)),
        ],
        out_specs=pl.BlockSpec((CE_BLK, D), lambda i: (i, 0)),
        out_shape=jax.ShapeDtypeStruct((E, D), f32),
    )(e, W_C, b_C.reshape(1, D))


def _zero_acc(sub, zbuf, acc):
    zero16 = jnp.zeros((16,), jnp.float32)

    def zrow(r, _):
        for k in range(D // 16):
            zbuf[r, pl.ds(k * 16, 16)] = zero16
        return 0

    lax.fori_loop(0, ZROWS, zrow, 0)
    for t in range(N_PAD // NSUB // ZROWS):
        row0 = pl.multiple_of(sub * (N_PAD // NSUB) + t * ZROWS, 8)
        pltpu.sync_copy(zbuf, acc.at[pl.ds(row0, ZROWS)])
    plsc.subcore_barrier()


def _publish_acc(core, sub, acc, out_hbm):
    plsc.subcore_barrier()
    rows = pl.multiple_of(sub * (N_PAD // NSUB), 8)
    pltpu.sync_copy(
        acc.at[pl.ds(rows, N_PAD // NSUB)],
        out_hbm.at[pl.ds(pl.multiple_of(core * N_PAD, 8) + rows,
                         N_PAD // NSUB)])


def _edge_num_body(src_hbm, dst_hbm, dh_tab, bh_tab, eh_tab, ce_hbm,
                   eij_hbm, num_hbm,
                   src_v, dst_v, dh_buf, bh_buf, eh_buf, ce_buf,
                   zbuf, acc, sem1, sem2, sem3):
    c = lax.axis_index("c")
    s = lax.axis_index("s")
    w = c * NSUB + s
    _zero_acc(s, zbuf, acc)

    def block(i, _):
        base = pl.multiple_of(w * E_PER_W + i * EB, 8)
        pltpu.sync_copy(src_hbm.at[pl.ds(base, EB)], src_v)
        pltpu.sync_copy(dst_hbm.at[pl.ds(base, EB)], dst_v)
        g1 = pltpu.async_copy(dh_tab.at[src_v], dh_buf, sem1)
        g2 = pltpu.async_copy(eh_tab.at[dst_v], eh_buf, sem2)
        g3 = pltpu.async_copy(bh_tab.at[src_v], bh_buf, sem3)
        pltpu.sync_copy(ce_hbm.at[pl.ds(base, EB)], ce_buf)
        g1.wait()
        g2.wait()
        g3.wait()

        def row(r, carry):
            for k in range(D // 16):
                sl = pl.ds(k * 16, 16)
                eij = ce_buf[r, sl] + dh_buf[r, sl] + eh_buf[r, sl]
                ce_buf[r, sl] = eij
                sig = 1.0 / (1.0 + jnp.exp(-eij))
                eh_buf[r, sl] = sig * bh_buf[r, sl]
            return carry

        lax.fori_loop(0, EB, row, 0)
        pltpu.sync_copy(ce_buf, eij_hbm.at[pl.ds(base, EB)])
        pltpu.sync_copy(eh_buf, acc.at[dst_v], add=True)
        return 0

    lax.fori_loop(0, NBLK_W, block, 0)
    _publish_acc(c, s, acc, num_hbm)


def _edge_num(src, dst, Dh, Bh, Eh, Ce):
    f32 = jnp.float32
    i32 = jnp.int32
    mesh = plsc.VectorSubcoreMesh(core_axis_name="c", subcore_axis_name="s")
    kern = pl.kernel(
        _edge_num_body,
        out_type=[
            jax.ShapeDtypeStruct((E, D), f32),
            jax.ShapeDtypeStruct((2 * N_PAD, D), f32),
        ],
        mesh=mesh,
        scratch_types=[
            pltpu.VMEM((EB,), i32),
            pltpu.VMEM((EB,), i32),
            pltpu.VMEM((EB, D), f32),
            pltpu.VMEM((EB, D), f32),
            pltpu.VMEM((EB, D), f32),
            pltpu.VMEM((EB, D), f32),
            pltpu.VMEM((ZROWS, D), f32),
            pltpu.VMEM_SHARED((N_PAD, D), f32),
            pltpu.SemaphoreType.DMA,
            pltpu.SemaphoreType.DMA,
            pltpu.SemaphoreType.DMA,
        ],
    )
    return kern(src, dst, Dh, Bh, Eh, Ce)


def _den_body(dst_hbm, eij_hbm, den_hbm,
              dst_v, eij_buf, zbuf, acc):
    c = lax.axis_index("c")
    s = lax.axis_index("s")
    w = c * NSUB + s
    _zero_acc(s, zbuf, acc)

    def block(i, _):
        base = pl.multiple_of(w * E_PER_W + i * EB, 8)
        pltpu.sync_copy(dst_hbm.at[pl.ds(base, EB)], dst_v)
        pltpu.sync_copy(eij_hbm.at[pl.ds(base, EB)], eij_buf)

        def row(r, carry):
            for k in range(D // 16):
                sl = pl.ds(k * 16, 16)
                eij_buf[r, sl] = 1.0 / (1.0 + jnp.exp(-eij_buf[r, sl]))
            return carry

        lax.fori_loop(0, EB, row, 0)
        pltpu.sync_copy(eij_buf, acc.at[dst_v], add=True)
        return 0

    lax.fori_loop(0, NBLK_W, block, 0)
    _publish_acc(c, s, acc, den_hbm)


def _den_scatter(dst, Eij):
    f32 = jnp.float32
    i32 = jnp.int32
    mesh = plsc.VectorSubcoreMesh(core_axis_name="c", subcore_axis_name="s")
    kern = pl.kernel(
        _den_body,
        out_type=jax.ShapeDtypeStruct((2 * N_PAD, D), f32),
        mesh=mesh,
        scratch_types=[
            pltpu.VMEM((EB,), i32),
            pltpu.VMEM((EB, D), f32),
            pltpu.VMEM((ZROWS, D), f32),
            pltpu.VMEM_SHARED((N_PAD, D), f32),
        ],
    )
    return kern(dst, Eij)


EST_BLK = 2000


def _ebn_body(eij_ref, ge_ref, be_ref, ebn_ref, acc_ref):
    i = pl.program_id(0)
    x = eij_ref[...]
    s = jnp.sum(x, axis=0, keepdims=True)
    q = jnp.sum(x * x, axis=0, keepdims=True)
    part = jnp.concatenate([s, q, jnp.zeros((6, D), jnp.float32)], axis=0)

    @pl.when(i == 0)
    def _():
        acc_ref[...] = part

    @pl.when(i > 0)
    def _():
        acc_ref[...] = acc_ref[...] + part

    @pl.when(i == E // EST_BLK - 1)
    def _():
        inv_e = 1.0 / E
        mu_e = acc_ref[0:1, :] * inv_e
        var_e = acc_ref[1:2, :] * inv_e - mu_e * mu_e
        sc_e = ge_ref[...] * jax.lax.rsqrt(var_e + EPS_BN)
        sh_e = be_ref[...] - mu_e * sc_e
        pad = jnp.zeros((6, D), jnp.float32)
        ebn_ref[...] = jnp.concatenate([sc_e, sh_e, pad], axis=0)


def _ebn_reduce(Eij, gamma_e, beta_e):
    return pl.pallas_call(
        _ebn_body,
        grid=(E // EST_BLK,),
        in_specs=[
            pl.BlockSpec((EST_BLK, D), lambda i: (i, 0)),
            pl.BlockSpec((1, D), lambda i: (0, 0)),
            pl.BlockSpec((1, D), lambda i: (0, 0)),
        ],
        out_specs=pl.BlockSpec((8, D), lambda i: (0, 0)),
        out_shape=jax.ShapeDtypeStruct((8, D), jnp.float32),
        scratch_shapes=[pltpu.VMEM((8, D), jnp.float32)],
    )(Eij, gamma_e.reshape(1, D), beta_e.reshape(1, D))


def _hpath_body(h_ref, ah_ref, num_ref, den_ref, gh_ref, bh_ref, hout_ref):
    num = num_ref[:N, :] + num_ref[N_PAD:N_PAD + N, :]
    den = den_ref[:N, :] + den_ref[N_PAD:N_PAD + N, :]
    h_new = ah_ref[...] + num / (den + EPS_DEN)
    mu = jnp.mean(h_new, axis=0, keepdims=True)
    var = jnp.mean(h_new * h_new, axis=0, keepdims=True) - mu * mu
    y = (h_new - mu) * jax.lax.rsqrt(var + EPS_BN) * gh_ref[...] + bh_ref[...]
    hout_ref[...] = h_ref[...] + jnp.maximum(y, 0.0)


def _hpath(h, Ah, num, den, gamma_h, beta_h):
    f32 = jnp.float32
    return pl.pallas_call(
        _hpath_body,
        out_shape=jax.ShapeDtypeStruct((N, D), f32),
    )(h, Ah, num, den, gamma_h.reshape(1, D), beta_h.reshape(1, D))


EO_BLK = 2000


def _epath_body(e_ref, eij_ref, ebn_ref, eout_ref):
    sc_e = ebn_ref[0:1, :]
    sh_e = ebn_ref[1:2, :]
    y = jnp.maximum(eij_ref[...] * sc_e + sh_e, 0.0)
    eout_ref[...] = e_ref[...] + y


def _epath(e, Eij, ebn):
    f32 = jnp.float32
    grid = E // EO_BLK
    return pl.pallas_call(
        _epath_body,
        grid=(grid,),
        in_specs=[
            pl.BlockSpec((EO_BLK, D), lambda i: (i, 0)),
            pl.BlockSpec((EO_BLK, D), lambda i: (i, 0)),
            pl.BlockSpec((8, D), lambda i: (0, 0)),
        ],
        out_specs=pl.BlockSpec((EO_BLK, D), lambda i: (i, 0)),
        out_shape=jax.ShapeDtypeStruct((E, D), f32),
    )(e, Eij, ebn)


def kernel(h, e, edge_index, W_A, b_A, W_B, b_B, W_C, b_C, W_D, b_D, W_E,
           b_E, gamma_h, beta_h, gamma_e, beta_e):
    src = edge_index[0]
    dst = edge_index[1]

    Ah, Eh, Dh, Bh = _node_linears(h, W_A, b_A, W_B, b_B, W_D, b_D, W_E, b_E)
    Ce = _ce_linear(e, W_C, b_C)

    Eij, num = _edge_num(src, dst, Dh, Bh, Eh, Ce)
    ebn = _ebn_reduce(Eij, gamma_e, beta_e)
    e_out = _epath(e, Eij, ebn)
    den = _den_scatter(dst, Eij)

    h_out = _hpath(h, Ah, num, den, gamma_h, beta_h)
    return (h_out, e_out)


# speedup vs baseline: 1.0548x; 1.0548x over previous
import jax
import jax.numpy as jnp
from jax import lax
from jax.experimental import pallas as pl
from jax.experimental.pallas import tpu as pltpu
from jax.experimental.pallas import tpu_sc as plsc

N = 10000
E = 320000
D = 128
NSUB = 16
EB = 80
EPS_DEN = 1e-6
EPS_BN = 1e-5

NW = 2 * NSUB
E_PER_W = E // NW
NBLK_W = E_PER_W // EB
N_PAD = 10240
ZROWS = 40


def _node_linear_body(h_ref, wa_ref, ba_ref, wb_ref, bb_ref, wd_ref,
                      bd_ref, we_ref, be_ref,
                      ah_ref, eh_ref, dh_ref, bh_ref):
    hv = h_ref[...]
    f32 = jnp.float32
    ah_ref[...] = jnp.dot(hv, wa_ref[...],
                          preferred_element_type=f32) + ba_ref[...]
    eh_ref[...] = jnp.dot(hv, we_ref[...],
                          preferred_element_type=f32) + be_ref[...]
    dh_ref[...] = jnp.dot(hv, wd_ref[...],
                          preferred_element_type=f32) + bd_ref[...]
    bh_ref[...] = jnp.dot(hv, wb_ref[...],
                          preferred_element_type=f32) + bb_ref[...]


def _node_linears(h, W_A, b_A, W_B, b_B, W_D, b_D, W_E, b_E):
    f32 = jnp.float32
    return pl.pallas_call(
        _node_linear_body,
        out_shape=[
            jax.ShapeDtypeStruct((N, D), f32),
            jax.ShapeDtypeStruct((N, D), f32),
            jax.ShapeDtypeStruct((N, D), f32),
            jax.ShapeDtypeStruct((N, D), f32),
        ],
    )(h, W_A, b_A.reshape(1, D), W_B, b_B.reshape(1, D), W_D,
      b_D.reshape(1, D), W_E, b_E.reshape(1, D))


CE_BLK = 2000


def _ce_body(e_ref, wc_ref, bc_ref, ce_ref):
    ce_ref[...] = jnp.dot(e_ref[...], wc_ref[...],
                          preferred_element_type=jnp.float32) + bc_ref[...]


def _ce_linear(e, W_C, b_C):
    f32 = jnp.float32
    grid = E // CE_BLK
    return pl.pallas_call(
        _ce_body,
        grid=(grid,),
        in_specs=[
            pl.BlockSpec((CE_BLK, D), lambda i: (i, 0)),
            pl.BlockSpec((D, D), lambda i: (0, 0)),
            pl.BlockSpec((1, D), lambda i: (0, 0)),
        ],
        out_specs=pl.BlockSpec((CE_BLK, D), lambda i: (i, 0)),
        out_shape=jax.ShapeDtypeStruct((E, D), f32),
    )(e, W_C, b_C.reshape(1, D))


def _zero_acc(sub, zbuf, acc):
    zero16 = jnp.zeros((16,), jnp.float32)

    def zrow(r, _):
        for k in range(D // 16):
            zbuf[r, pl.ds(k * 16, 16)] = zero16
        return 0

    lax.fori_loop(0, ZROWS, zrow, 0)
    for t in range(N_PAD // NSUB // ZROWS):
        row0 = pl.multiple_of(sub * (N_PAD // NSUB) + t * ZROWS, 8)
        pltpu.sync_copy(zbuf, acc.at[pl.ds(row0, ZROWS)])
    plsc.subcore_barrier()


def _publish_acc(core, sub, acc, out_hbm):
    plsc.subcore_barrier()
    rows = pl.multiple_of(sub * (N_PAD // NSUB), 8)
    pltpu.sync_copy(
        acc.at[pl.ds(rows, N_PAD // NSUB)],
        out_hbm.at[pl.ds(pl.multiple_of(core * N_PAD, 8) + rows,
                         N_PAD // NSUB)])


def _edge_num_body(src_hbm, dst_hbm, dh_tab, bh_tab, eh_tab, ce_hbm,
                   eij_hbm, sig_hbm, num_hbm,
                   src_v, dst_v, dh_buf, bh_buf, eh_buf, ce_buf,
                   zbuf, acc, sem1, sem2, sem3):
    c = lax.axis_index("c")
    s = lax.axis_index("s")
    w = c * NSUB + s
    _zero_acc(s, zbuf, acc)

    def block(i, _):
        base = pl.multiple_of(w * E_PER_W + i * EB, 8)
        pltpu.sync_copy(src_hbm.at[pl.ds(base, EB)], src_v)
        pltpu.sync_copy(dst_hbm.at[pl.ds(base, EB)], dst_v)
        g1 = pltpu.async_copy(dh_tab.at[src_v], dh_buf, sem1)
        g2 = pltpu.async_copy(eh_tab.at[dst_v], eh_buf, sem2)
        g3 = pltpu.async_copy(bh_tab.at[src_v], bh_buf, sem3)
        pltpu.sync_copy(ce_hbm.at[pl.ds(base, EB)], ce_buf)
        g1.wait()
        g2.wait()
        g3.wait()

        def row(r, carry):
            for k in range(D // 16):
                sl = pl.ds(k * 16, 16)
                eij = ce_buf[r, sl] + dh_buf[r, sl] + eh_buf[r, sl]
                ce_buf[r, sl] = eij
                sig = 1.0 / (1.0 + jnp.exp(-eij))
                eh_buf[r, sl] = sig * bh_buf[r, sl]
                bh_buf[r, sl] = sig
            return carry

        lax.fori_loop(0, EB, row, 0)
        pltpu.sync_copy(ce_buf, eij_hbm.at[pl.ds(base, EB)])
        pltpu.sync_copy(bh_buf, sig_hbm.at[pl.ds(base, EB)])
        pltpu.sync_copy(eh_buf, acc.at[dst_v], add=True)
        return 0

    lax.fori_loop(0, NBLK_W, block, 0)
    _publish_acc(c, s, acc, num_hbm)


def _edge_num(src, dst, Dh, Bh, Eh, Ce):
    f32 = jnp.float32
    i32 = jnp.int32
    mesh = plsc.VectorSubcoreMesh(core_axis_name="c", subcore_axis_name="s")
    kern = pl.kernel(
        _edge_num_body,
        out_type=[
            jax.ShapeDtypeStruct((E, D), f32),
            jax.ShapeDtypeStruct((E, D), f32),
            jax.ShapeDtypeStruct((2 * N_PAD, D), f32),
        ],
        mesh=mesh,
        scratch_types=[
            pltpu.VMEM((EB,), i32),
            pltpu.VMEM((EB,), i32),
            pltpu.VMEM((EB, D), f32),
            pltpu.VMEM((EB, D), f32),
            pltpu.VMEM((EB, D), f32),
            pltpu.VMEM((EB, D), f32),
            pltpu.VMEM((ZROWS, D), f32),
            pltpu.VMEM_SHARED((N_PAD, D), f32),
            pltpu.SemaphoreType.DMA,
            pltpu.SemaphoreType.DMA,
            pltpu.SemaphoreType.DMA,
        ],
    )
    return kern(src, dst, Dh, Bh, Eh, Ce)


def _den_body(dst_hbm, sig_hbm, den_hbm,
              dst_v, sig_buf, zbuf, acc):
    c = lax.axis_index("c")
    s = lax.axis_index("s")
    w = c * NSUB + s
    _zero_acc(s, zbuf, acc)

    def block(i, _):
        base = pl.multiple_of(w * E_PER_W + i * EB, 8)
        pltpu.sync_copy(dst_hbm.at[pl.ds(base, EB)], dst_v)
        pltpu.sync_copy(sig_hbm.at[pl.ds(base, EB)], sig_buf)
        pltpu.sync_copy(sig_buf, acc.at[dst_v], add=True)
        return 0

    lax.fori_loop(0, NBLK_W, block, 0)
    _publish_acc(c, s, acc, den_hbm)


def _den_scatter(dst, Sig):
    f32 = jnp.float32
    i32 = jnp.int32
    mesh = plsc.VectorSubcoreMesh(core_axis_name="c", subcore_axis_name="s")
    kern = pl.kernel(
        _den_body,
        out_type=jax.ShapeDtypeStruct((2 * N_PAD, D), f32),
        mesh=mesh,
        scratch_types=[
            pltpu.VMEM((EB,), i32),
            pltpu.VMEM((EB, D), f32),
            pltpu.VMEM((ZROWS, D), f32),
            pltpu.VMEM_SHARED((N_PAD, D), f32),
        ],
    )
    return kern(dst, Sig)


EST_BLK = 2000


def _ebn_body(eij_ref, ge_ref, be_ref, ebn_ref, acc_ref):
    i = pl.program_id(0)
    x = eij_ref[...]
    s = jnp.sum(x, axis=0, keepdims=True)
    q = jnp.sum(x * x, axis=0, keepdims=True)
    part = jnp.concatenate([s, q, jnp.zeros((6, D), jnp.float32)], axis=0)

    @pl.when(i == 0)
    def _():
        acc_ref[...] = part

    @pl.when(i > 0)
    def _():
        acc_ref[...] = acc_ref[...] + part

    @pl.when(i == E // EST_BLK - 1)
    def _():
        inv_e = 1.0 / E
        mu_e = acc_ref[0:1, :] * inv_e
        var_e = acc_ref[1:2, :] * inv_e - mu_e * mu_e
        sc_e = ge_ref[...] * jax.lax.rsqrt(var_e + EPS_BN)
        sh_e = be_ref[...] - mu_e * sc_e
        pad = jnp.zeros((6, D), jnp.float32)
        ebn_ref[...] = jnp.concatenate([sc_e, sh_e, pad], axis=0)


def _ebn_reduce(Eij, gamma_e, beta_e):
    return pl.pallas_call(
        _ebn_body,
        grid=(E // EST_BLK,),
        in_specs=[
            pl.BlockSpec((EST_BLK, D), lambda i: (i, 0)),
            pl.BlockSpec((1, D), lambda i: (0, 0)),
            pl.BlockSpec((1, D), lambda i: (0, 0)),
        ],
        out_specs=pl.BlockSpec((8, D), lambda i: (0, 0)),
        out_shape=jax.ShapeDtypeStruct((8, D), jnp.float32),
        scratch_shapes=[pltpu.VMEM((8, D), jnp.float32)],
    )(Eij, gamma_e.reshape(1, D), beta_e.reshape(1, D))


def _hpath_body(h_ref, ah_ref, num_ref, den_ref, gh_ref, bh_ref, hout_ref):
    num = num_ref[:N, :] + num_ref[N_PAD:N_PAD + N, :]
    den = den_ref[:N, :] + den_ref[N_PAD:N_PAD + N, :]
    h_new = ah_ref[...] + num / (den + EPS_DEN)
    mu = jnp.mean(h_new, axis=0, keepdims=True)
    var = jnp.mean(h_new * h_new, axis=0, keepdims=True) - mu * mu
    y = (h_new - mu) * jax.lax.rsqrt(var + EPS_BN) * gh_ref[...] + bh_ref[...]
    hout_ref[...] = h_ref[...] + jnp.maximum(y, 0.0)


def _hpath(h, Ah, num, den, gamma_h, beta_h):
    f32 = jnp.float32
    return pl.pallas_call(
        _hpath_body,
        out_shape=jax.ShapeDtypeStruct((N, D), f32),
    )(h, Ah, num, den, gamma_h.reshape(1, D), beta_h.reshape(1, D))


EO_BLK = 2000


def _epath_body(e_ref, eij_ref, ebn_ref, eout_ref):
    sc_e = ebn_ref[0:1, :]
    sh_e = ebn_ref[1:2, :]
    y = jnp.maximum(eij_ref[...] * sc_e + sh_e, 0.0)
    eout_ref[...] = e_ref[...] + y


def _epath(e, Eij, ebn):
    f32 = jnp.float32
    grid = E // EO_BLK
    return pl.pallas_call(
        _epath_body,
        grid=(grid,),
        in_specs=[
            pl.BlockSpec((EO_BLK, D), lambda i: (i, 0)),
            pl.BlockSpec((EO_BLK, D), lambda i: (i, 0)),
            pl.BlockSpec((8, D), lambda i: (0, 0)),
        ],
        out_specs=pl.BlockSpec((EO_BLK, D), lambda i: (i, 0)),
        out_shape=jax.ShapeDtypeStruct((E, D), f32),
    )(e, Eij, ebn)


def kernel(h, e, edge_index, W_A, b_A, W_B, b_B, W_C, b_C, W_D, b_D, W_E,
           b_E, gamma_h, beta_h, gamma_e, beta_e):
    src = edge_index[0]
    dst = edge_index[1]

    Ah, Eh, Dh, Bh = _node_linears(h, W_A, b_A, W_B, b_B, W_D, b_D, W_E, b_E)
    Ce = _ce_linear(e, W_C, b_C)

    Eij, Sig, num = _edge_num(src, dst, Dh, Bh, Eh, Ce)
    ebn = _ebn_reduce(Eij, gamma_e, beta_e)
    e_out = _epath(e, Eij, ebn)
    den = _den_scatter(dst, Sig)

    h_out = _hpath(h, Ah, num, den, gamma_h, beta_h)
    return (h_out, e_out)
